# baseline (speedup 1.0000x reference)
import functools

import jax
import jax.numpy as jnp
from jax import lax
from jax.experimental import pallas as pl
from jax.experimental.pallas import tpu as pltpu
from jax.experimental.pallas import tpu_sc as plsc

NUM_EXPERTS = 16
BATCH = 128
TOP_K = 2
ROW_ELEMS = 8192 * 16
BKT = BATCH * TOP_K

C = 32
D = ROW_ELEMS // C
LOG2C = 5

NC = 2
NS = 16
NW = NC * NS
R = BKT * C // NW
BK_PER_W = BKT // NW
G = 8
NG = R // G
LANES = 16


def _body(table_hbm, idx_hbm, out_hbm, idx8, eidx, buf0, buf1, sem0, sem1):
    c = lax.axis_index("c")
    s = lax.axis_index("s")
    w = s * NC + c
    base = w * R

    pltpu.sync_copy(idx_hbm.at[pl.ds(w * BK_PER_W, BK_PER_W)], idx8)

    def expand(j, carry):
        r0 = j * LANES
        rv = r0 + lax.iota(jnp.int32, LANES)
        e = plsc.load_gather(idx8, [jnp.right_shift(rv, LOG2C)])
        eidx[pl.ds(r0, LANES)] = e * C + jnp.bitwise_and(rv, C - 1)
        return carry

    lax.fori_loop(0, R // LANES, expand, 0, unroll=True)

    def step2(i, carry):
        g = i * 2
        cpa = pltpu.async_copy(
            table_hbm.at[eidx.at[pl.ds(g * G, G)]], buf0, sem0)
        cpb = pltpu.async_copy(
            table_hbm.at[eidx.at[pl.ds((g + 1) * G, G)]], buf1, sem1)
        cpa.wait()
        pltpu.sync_copy(buf0, out_hbm.at[pl.ds(base + g * G, G)])
        cpb.wait()
        pltpu.sync_copy(buf1, out_hbm.at[pl.ds(base + (g + 1) * G, G)])
        return carry

    lax.fori_loop(0, NG // 2, step2, 0)


@jax.jit
def _gather_sc(indices_flat, table):
    kfn = pl.kernel(
        _body,
        out_type=jax.ShapeDtypeStruct((BKT * C, D), jnp.float32),
        mesh=plsc.VectorSubcoreMesh(
            core_axis_name="c", subcore_axis_name="s",
            num_cores=NC, num_subcores=NS),
        scratch_types=[
            pltpu.VMEM((BK_PER_W,), jnp.int32),
            pltpu.VMEM((R,), jnp.int32),
            pltpu.VMEM((G, D), jnp.float32),
            pltpu.VMEM((G, D), jnp.float32),
            pltpu.SemaphoreType.DMA,
            pltpu.SemaphoreType.DMA,
        ],
        compiler_params=pltpu.CompilerParams(needs_layout_passes=False),
    )
    return kfn(table, indices_flat)


def kernel(indices, A_experts):
    idx = indices.reshape(BKT).astype(jnp.int32)
    table = A_experts.reshape(NUM_EXPERTS * C, D)
    out = _gather_sc(idx, table)
    return out.reshape(BATCH, TOP_K, 8192, 16)


# speedup vs baseline: 1.0036x; 1.0036x over previous
import jax
import jax.numpy as jnp
from jax import lax
from jax.experimental import pallas as pl
from jax.experimental.pallas import tpu as pltpu
from jax.experimental.pallas import tpu_sc as plsc

NUM_EXPERTS = 16
BATCH = 128
TOP_K = 2
ROW_ELEMS = 8192 * 16
BKT = BATCH * TOP_K

C = 32
D = ROW_ELEMS // C
LOG2C = 5

NC = 2
NS = 16
NW = NC * NS
R = BKT * C // NW
BK_PER_W = BKT // NW
G = 8
NG = R // G
NBUF = 3
LANES = 16


def _body(table_hbm, idx_hbm, out_hbm, idx8, eidx, bufs, gsems, wsems):
    c = lax.axis_index("c")
    s = lax.axis_index("s")
    w = s * NC + c
    base = w * R

    pltpu.sync_copy(idx_hbm.at[pl.ds(w * BK_PER_W, BK_PER_W)],
                    idx8.at[pl.ds(0, BK_PER_W)])

    def expand(j, carry):
        r0 = j * LANES
        rv = r0 + lax.iota(jnp.int32, LANES)
        e = plsc.load_gather(idx8, [jnp.right_shift(rv, LOG2C)])
        eidx[pl.ds(r0, LANES)] = e * C + jnp.bitwise_and(rv, C - 1)
        return carry

    lax.fori_loop(0, R // LANES, expand, 0, unroll=True)

    gh = [None] * NG
    wh = [None] * NG
    for g in range(NG):
        slot = g % NBUF
        if g >= NBUF:
            wh[g - NBUF].wait()
        gh[g] = pltpu.async_copy(
            table_hbm.at[eidx.at[pl.ds(g * G, G)]], bufs[slot], gsems[slot])
        if g >= 1:
            gp = g - 1
            gh[gp].wait()
            wh[gp] = pltpu.async_copy(
                bufs[gp % NBUF], out_hbm.at[pl.ds(base + gp * G, G)],
                wsems[gp % NBUF])
    gh[NG - 1].wait()
    wh[NG - 1] = pltpu.async_copy(
        bufs[(NG - 1) % NBUF], out_hbm.at[pl.ds(base + (NG - 1) * G, G)],
        wsems[(NG - 1) % NBUF])
    for g in range(NG - NBUF, NG):
        wh[g].wait()


def _body_wrap(table_hbm, idx_hbm, out_hbm, idx8, eidx,
               b0, b1, b2, g0, g1, g2, w0, w1, w2):
    _body(table_hbm, idx_hbm, out_hbm, idx8, eidx,
          [b0, b1, b2], [g0, g1, g2], [w0, w1, w2])


@jax.jit
def _gather_sc(indices_flat, table):
    kfn = pl.kernel(
        _body_wrap,
        out_type=jax.ShapeDtypeStruct((BKT * C, D), jnp.float32),
        mesh=plsc.VectorSubcoreMesh(
            core_axis_name="c", subcore_axis_name="s",
            num_cores=NC, num_subcores=NS),
        scratch_types=(
            [pltpu.VMEM((LANES,), jnp.int32),
             pltpu.VMEM((R,), jnp.int32)]
            + [pltpu.VMEM((G, D), jnp.float32)] * NBUF
            + [pltpu.SemaphoreType.DMA] * (2 * NBUF)
        ),
        compiler_params=pltpu.CompilerParams(needs_layout_passes=False),
    )
    return kfn(table, indices_flat)


def kernel(indices, A_experts):
    idx = indices.reshape(BKT).astype(jnp.int32)
    table = A_experts.reshape(NUM_EXPERTS * C, D)
    out = _gather_sc(idx, table)
    return out.reshape(BATCH, TOP_K, 8192, 16)


# speedup vs baseline: 2.7062x; 2.6966x over previous
import jax
import jax.numpy as jnp
from jax import lax
from jax.experimental import pallas as pl
from jax.experimental.pallas import tpu as pltpu
from jax.experimental.pallas import tpu_sc as plsc

NUM_EXPERTS = 16
BATCH = 128
TOP_K = 2
D_STATE = 16
ROWS = 8192
BKT = BATCH * TOP_K

C = 32
CH = ROWS // C
SL = CH * D_STATE // 128
LOG2C = 5

NC = 2
NS = 16
NW = NC * NS
R = BKT * C // NW
BK_PER_W = BKT // NW
G = 8
NG = R // G
GPB = C // G
NBUF = 3
LANES = 16


def _body(table_hbm, idx_hbm, out_hbm, idx8, eidx, bufs, gsems, wsems):
    c = lax.axis_index("c")
    s = lax.axis_index("s")
    w = s * NC + c

    pltpu.sync_copy(idx_hbm.at[pl.ds(w * BK_PER_W, BK_PER_W)],
                    idx8.at[pl.ds(0, BK_PER_W)])

    def expand(j, carry):
        r0 = j * LANES
        rv = r0 + lax.iota(jnp.int32, LANES)
        e = plsc.load_gather(idx8, [jnp.right_shift(rv, LOG2C)])
        eidx[pl.ds(r0, LANES)] = e * C + jnp.bitwise_and(rv, C - 1)
        return carry

    lax.fori_loop(0, R // LANES, expand, 0, unroll=True)

    def write_dst(g):
        b = w * (BK_PER_W // TOP_K) + (g // GPB) // TOP_K
        k = (g // GPB) % TOP_K
        return out_hbm.at[b, k, pl.ds((g % GPB) * G, G)]

    gh = [None] * NG
    wh = [None] * NG
    for g in range(NG):
        slot = g % NBUF
        if g >= NBUF:
            wh[g - NBUF].wait()
        gh[g] = pltpu.async_copy(
            table_hbm.at[eidx.at[pl.ds(g * G, G)]], bufs[slot], gsems[slot])
        if g >= 1:
            gh[g - 1].wait()
            wh[g - 1] = pltpu.async_copy(
                bufs[(g - 1) % NBUF], write_dst(g - 1), wsems[(g - 1) % NBUF])
    gh[NG - 1].wait()
    wh[NG - 1] = pltpu.async_copy(
        bufs[(NG - 1) % NBUF], write_dst(NG - 1), wsems[(NG - 1) % NBUF])
    for g in range(NG - NBUF, NG):
        wh[g].wait()


def _body_wrap(table_hbm, idx_hbm, out_hbm, idx8, eidx,
               b0, b1, b2, g0, g1, g2, w0, w1, w2):
    _body(table_hbm, idx_hbm, out_hbm, idx8, eidx,
          [b0, b1, b2], [g0, g1, g2], [w0, w1, w2])


@jax.jit
def _gather_sc(indices_flat, table):
    kfn = pl.kernel(
        _body_wrap,
        out_type=jax.ShapeDtypeStruct((BATCH, TOP_K, C, SL, 128),
                                      jnp.float32),
        mesh=plsc.VectorSubcoreMesh(
            core_axis_name="c", subcore_axis_name="s",
            num_cores=NC, num_subcores=NS),
        scratch_types=(
            [pltpu.VMEM((LANES,), jnp.int32),
             pltpu.VMEM((R,), jnp.int32)]
            + [pltpu.VMEM((G, SL, 128), jnp.float32)] * NBUF
            + [pltpu.SemaphoreType.DMA] * (2 * NBUF)
        ),
        compiler_params=pltpu.CompilerParams(needs_layout_passes=False),
    )
    return kfn(table, indices_flat)


def kernel(indices, A_experts):
    idx = indices.reshape(BKT).astype(jnp.int32)
    table = A_experts.reshape(NUM_EXPERTS * C, SL, 128)
    out = _gather_sc(idx, table)
    return out.reshape(BATCH, TOP_K, ROWS, D_STATE)


# speedup vs baseline: 9.0502x; 3.3443x over previous
import jax
import jax.numpy as jnp
from jax import lax
from jax.experimental import pallas as pl
from jax.experimental.pallas import tpu as pltpu
from jax.experimental.pallas import tpu_sc as plsc

NUM_EXPERTS = 16
BATCH = 128
TOP_K = 2
D_STATE = 16
ROWS = 8192
BKT = BATCH * TOP_K

NC = 2
NS = 16
NW = NC * NS
BK_PER_W = BKT // NW
B_PER_W = BK_PER_W // TOP_K

DT = 8
RH = ROWS // 2
CPB = (D_STATE // DT) * (ROWS // RH)
NG = BK_PER_W * CPB
NBUF = 3
LANES = 16


def _body(table_hbm, idx_hbm, out_hbm, idx8, bufs, gsems, wsems):
    c = lax.axis_index("c")
    s = lax.axis_index("s")
    w = s * NC + c

    pltpu.sync_copy(idx_hbm.at[pl.ds(w * BK_PER_W, BK_PER_W)],
                    idx8.at[pl.ds(0, BK_PER_W)])
    ids = idx8[...]

    def src(g, e):
        dt = (g % CPB) // 2
        rh = g % 2
        return table_hbm.at[e, pl.ds(dt * DT, DT), pl.ds(rh * RH, RH)]

    def dst(g):
        b = w * B_PER_W + (g // CPB) // TOP_K
        k = (g // CPB) % TOP_K
        dt = (g % CPB) // 2
        rh = g % 2
        return out_hbm.at[b, k, pl.ds(dt * DT, DT), pl.ds(rh * RH, RH)]

    gh = [None] * NG
    wh = [None] * NG
    for g in range(NG):
        slot = g % NBUF
        if g >= NBUF:
            wh[g - NBUF].wait()
        gh[g] = pltpu.async_copy(src(g, ids[g // CPB]), bufs[slot],
                                 gsems[slot])
        if g >= 1:
            gh[g - 1].wait()
            wh[g - 1] = pltpu.async_copy(
                bufs[(g - 1) % NBUF], dst(g - 1), wsems[(g - 1) % NBUF])
    gh[NG - 1].wait()
    wh[NG - 1] = pltpu.async_copy(
        bufs[(NG - 1) % NBUF], dst(NG - 1), wsems[(NG - 1) % NBUF])
    for g in range(NG - NBUF, NG):
        wh[g].wait()


def _body_wrap(table_hbm, idx_hbm, out_hbm, idx8,
               b0, b1, b2, g0, g1, g2, w0, w1, w2):
    _body(table_hbm, idx_hbm, out_hbm, idx8,
          [b0, b1, b2], [g0, g1, g2], [w0, w1, w2])


@jax.jit
def _gather_sc(indices_flat, table_t):
    kfn = pl.kernel(
        _body_wrap,
        out_type=jax.ShapeDtypeStruct((BATCH, TOP_K, D_STATE, ROWS),
                                      jnp.float32),
        mesh=plsc.VectorSubcoreMesh(
            core_axis_name="c", subcore_axis_name="s",
            num_cores=NC, num_subcores=NS),
        scratch_types=(
            [pltpu.VMEM((LANES,), jnp.int32)]
            + [pltpu.VMEM((DT, RH), jnp.float32)] * NBUF
            + [pltpu.SemaphoreType.DMA] * (2 * NBUF)
        ),
        compiler_params=pltpu.CompilerParams(needs_layout_passes=False),
    )
    return kfn(table_t, indices_flat)


def kernel(indices, A_experts):
    idx = indices.reshape(BKT).astype(jnp.int32)
    table_t = jnp.transpose(A_experts, (0, 2, 1))
    out_t = _gather_sc(idx, table_t)
    return jnp.transpose(out_t, (0, 1, 3, 2))


# speedup vs baseline: 11.2002x; 1.2376x over previous
import jax
import jax.numpy as jnp
from jax import lax
from jax.experimental import pallas as pl
from jax.experimental.pallas import tpu as pltpu
from jax.experimental.pallas import tpu_sc as plsc

NUM_EXPERTS = 16
BATCH = 128
TOP_K = 2
D_STATE = 16
ROWS = 8192
BKT = BATCH * TOP_K

NC = 2
NS = 16
DT = D_STATE // NC
BK_PER_S = BKT // NS
LANES = 16


def _body(table_hbm, idx_hbm, out_hbm, idx16, shared, sem):
    c = lax.axis_index("c")
    s = lax.axis_index("s")

    pltpu.sync_copy(table_hbm.at[s, pl.ds(c * DT, DT)], shared.at[s])

    pltpu.sync_copy(idx_hbm.at[pl.ds(s * BK_PER_S, BK_PER_S)], idx16)
    ids = idx16[...]

    plsc.subcore_barrier()

    copies = []
    for j in range(BK_PER_S):
        b = s * (BK_PER_S // TOP_K) + j // TOP_K
        k = j % TOP_K
        copies.append(pltpu.async_copy(
            shared.at[ids[j]],
            out_hbm.at[b, k, pl.ds(c * DT, DT)],
            sem))
    for cp in copies:
        cp.wait()


@jax.jit
def _gather_sc(indices_flat, table_t):
    kfn = pl.kernel(
        _body,
        out_type=jax.ShapeDtypeStruct((BATCH, TOP_K, D_STATE, ROWS),
                                      jnp.float32),
        mesh=plsc.VectorSubcoreMesh(
            core_axis_name="c", subcore_axis_name="s",
            num_cores=NC, num_subcores=NS),
        scratch_types=[
            pltpu.VMEM((LANES,), jnp.int32),
            pltpu.VMEM_SHARED((NUM_EXPERTS, DT, ROWS), jnp.float32),
            pltpu.SemaphoreType.DMA,
        ],
        compiler_params=pltpu.CompilerParams(needs_layout_passes=False),
    )
    return kfn(table_t, indices_flat)


def kernel(indices, A_experts):
    idx = indices.reshape(BKT).astype(jnp.int32)
    table_t = jnp.transpose(A_experts, (0, 2, 1))
    out_t = _gather_sc(idx, table_t)
    return jnp.transpose(out_t, (0, 1, 3, 2))
